# baseline (speedup 1.0000x reference)
import functools

import jax
import jax.numpy as jnp
from jax import lax
from jax.experimental import pallas as pl
from jax.experimental.pallas import tpu as pltpu
from jax.experimental.pallas import tpu_sc as plsc

EMBED_DIM = 64
N_ACTION = 2
N_CORES = 2
N_SUBCORES = 16
NW = N_CORES * N_SUBCORES
LANES = 16
ROW = 128
RBLK = 64
CBLK = RBLK * ROW
COLS_PER_TEC = 16384
SC_COLS = COLS_PER_TEC * NW
CCH = 512
N_CH = COLS_PER_TEC // CCH
CHUNK = 256


def _proj_body(w_ref, b_ref, x_ref, o0_ref, o1_ref):
    y = lax.dot_general(
        w_ref[...], x_ref[...],
        (((1,), (0,)), ((), ())),
        preferred_element_type=jnp.float32,
        precision=lax.Precision.HIGHEST,
    ) + b_ref[...]
    o0_ref[...] = y[0:1, :].reshape(RBLK, ROW)
    o1_ref[...] = y[1:2, :].reshape(RBLK, ROW)


@functools.lru_cache(maxsize=None)
def _make_tc_proj(n_rows: int):
    n_prows = (n_rows + ROW - 1) // ROW - SC_COLS // ROW
    blk0 = SC_COLS // CBLK
    grid = (n_prows * ROW + CBLK - 1) // CBLK
    return pl.pallas_call(
        _proj_body,
        grid=(grid,),
        in_specs=[
            pl.BlockSpec((N_ACTION, EMBED_DIM), lambda i: (0, 0)),
            pl.BlockSpec((N_ACTION, 1), lambda i: (0, 0)),
            pl.BlockSpec((EMBED_DIM, CBLK), lambda i: (0, blk0 + i)),
        ],
        out_specs=[
            pl.BlockSpec((RBLK, ROW), lambda i: (i, 0)),
            pl.BlockSpec((RBLK, ROW), lambda i: (i, 0)),
        ],
        out_shape=[
            jax.ShapeDtypeStruct((n_prows, ROW), jnp.float32),
            jax.ShapeDtypeStruct((n_prows, ROW), jnp.float32),
        ],
    )


@functools.lru_cache(maxsize=None)
def _make_sc_proj():
    mesh = plsc.VectorSubcoreMesh(core_axis_name="c", subcore_axis_name="s")

    @functools.partial(
        pl.kernel,
        mesh=mesh,
        out_type=[
            jax.ShapeDtypeStruct((SC_COLS,), jnp.float32),
            jax.ShapeDtypeStruct((SC_COLS,), jnp.float32),
        ],
        scratch_types=[
            pltpu.VMEM((EMBED_DIM // 8, 8, CCH), jnp.float32),
            pltpu.VMEM((EMBED_DIM // 8, 8, CCH), jnp.float32),
            pltpu.VMEM((256,), jnp.float32),
            pltpu.VMEM((COLS_PER_TEC,), jnp.float32),
            pltpu.VMEM((COLS_PER_TEC,), jnp.float32),
            pltpu.SemaphoreType.DMA,
            pltpu.SemaphoreType.DMA,
        ],
        compiler_params=pltpu.CompilerParams(
            use_tc_tiling_on_sc=True, needs_layout_passes=False
        ),
    )
    def sc_proj(table_hbm, wb_hbm, y0_hbm, y1_hbm,
                buf_a, buf_b, wb_v, out0_v, out1_v, sem_a, sem_b):
        wid = lax.axis_index("s") * N_CORES + lax.axis_index("c")
        cbase = wid * COLS_PER_TEC
        pltpu.sync_copy(wb_hbm, wb_v)
        w_vecs = [
            [wb_v[pl.ds(a * EMBED_DIM + k * LANES, LANES)] for k in range(4)]
            for a in range(N_ACTION)
        ]
        bias_vec = wb_v[pl.ds(2 * EMBED_DIM, LANES)]
        bufs = (buf_a, buf_b)
        sems = (sem_a, sem_b)

        for b in range(2):
            pltpu.async_copy(
                table_hbm.at[:, :, pl.ds(cbase + b * CCH, CCH)],
                bufs[b], sems[b],
            )

        def outer(c2, _):
            for b in range(2):
                c = c2 * 2 + b
                buf, sem = bufs[b], sems[b]
                pltpu.make_async_copy(
                    table_hbm.at[:, :, pl.ds(0, CCH)], buf, sem
                ).wait()

                def blk_body(jb, _):
                    off = jb * LANES
                    acc = [jnp.full((LANES,), bias_vec[a], jnp.float32)
                           for a in range(N_ACTION)]
                    for d in range(EMBED_DIM):
                        val = buf[d // 8, d % 8, pl.ds(off, LANES)]
                        for a in range(N_ACTION):
                            acc[a] = acc[a] + val * w_vecs[a][d // 16][d % 16]
                    out0_v[pl.ds(c * CCH + off, LANES)] = acc[0]
                    out1_v[pl.ds(c * CCH + off, LANES)] = acc[1]
                    return 0

                lax.fori_loop(0, CCH // LANES, blk_body, 0)

                @pl.when(c + 2 < N_CH)
                def _():
                    pltpu.async_copy(
                        table_hbm.at[:, :, pl.ds(cbase + (c + 2) * CCH, CCH)],
                        buf, sem,
                    )
            return 0

        lax.fori_loop(0, N_CH // 2, outer, 0)
        pltpu.sync_copy(out0_v, y0_hbm.at[pl.ds(cbase, COLS_PER_TEC)])
        pltpu.sync_copy(out1_v, y1_hbm.at[pl.ds(cbase, COLS_PER_TEC)])

    return sc_proj


@functools.lru_cache(maxsize=None)
def _make_lookup(batch: int, n_prows: int):
    assert batch % (CHUNK * NW) == 0
    b_per_w = batch // NW
    n_chunks = b_per_w // CHUNK
    mesh = plsc.VectorSubcoreMesh(core_axis_name="c", subcore_axis_name="s")

    @functools.partial(
        pl.kernel,
        mesh=mesh,
        out_type=jax.ShapeDtypeStruct((N_ACTION, batch), jnp.float32),
        scratch_types=[
            pltpu.VMEM((b_per_w,), jnp.int32),
            pltpu.VMEM((b_per_w,), jnp.int32),
            pltpu.VMEM((CHUNK, ROW), jnp.float32),
            pltpu.VMEM((CHUNK, ROW), jnp.float32),
            pltpu.VMEM((b_per_w,), jnp.float32),
            pltpu.VMEM((b_per_w,), jnp.float32),
            pltpu.SemaphoreType.DMA,
        ],
        compiler_params=pltpu.CompilerParams(
            use_tc_tiling_on_sc=True, needs_layout_passes=False
        ),
    )
    def lookup(idx_hbm, y0_hbm, y1_hbm, out_hbm,
               idx_v, row_v, g0_v, g1_v, out0_v, out1_v, sem):
        wid = lax.axis_index("s") * N_CORES + lax.axis_index("c")
        base = wid * b_per_w
        pltpu.sync_copy(idx_hbm.at[pl.ds(base, b_per_w)], idx_v)

        def split_body(k, _):
            v = idx_v[pl.ds(k * LANES, LANES)]
            row_v[pl.ds(k * LANES, LANES)] = v >> 7
            return 0

        lax.fori_loop(0, b_per_w // LANES, split_body, 0)
        lane_iota = lax.iota(jnp.int32, LANES)

        def chunk_body(c, _):
            rows = row_v.at[pl.ds(c * CHUNK, CHUNK)]
            cp0 = pltpu.async_copy(y0_hbm.at[rows], g0_v, sem)
            cp1 = pltpu.async_copy(y1_hbm.at[rows], g1_v, sem)
            cp0.wait()
            cp1.wait()

            def blk_body(jb, _):
                iv = idx_v[pl.ds(c * CHUNK + jb * LANES, LANES)]
                slot = jb * LANES + lane_iota
                col = iv & (ROW - 1)
                v0 = plsc.load_gather(g0_v, [slot, col])
                v1 = plsc.load_gather(g1_v, [slot, col])
                out0_v[pl.ds(c * CHUNK + jb * LANES, LANES)] = v0
                out1_v[pl.ds(c * CHUNK + jb * LANES, LANES)] = v1
                return 0

            lax.fori_loop(0, CHUNK // LANES, blk_body, 0)
            return 0

        lax.fori_loop(0, n_chunks, chunk_body, 0)
        pltpu.sync_copy(out0_v, out_hbm.at[0, pl.ds(base, b_per_w)])
        pltpu.sync_copy(out1_v, out_hbm.at[1, pl.ds(base, b_per_w)])

    return lookup


def kernel(user_idx, table, W, b):
    batch = user_idx.shape[0]
    n_rows = table.shape[0]
    n_prows = (n_rows + ROW - 1) // ROW
    table_t = table.T
    table_t3 = table_t.reshape(EMBED_DIM // 8, 8, n_rows)
    wb = jnp.zeros((256,), jnp.float32)
    wb = wb.at[: N_ACTION * EMBED_DIM].set(W.reshape(-1))
    wb = wb.at[2 * EMBED_DIM : 2 * EMBED_DIM + N_ACTION].set(b)

    y0_sc, y1_sc = _make_sc_proj()(table_t3, wb)
    y0_tc, y1_tc = _make_tc_proj(n_rows)(W, b.reshape(N_ACTION, 1), table_t)
    y0 = jnp.concatenate([y0_sc.reshape(SC_COLS // ROW, ROW), y0_tc], axis=0)
    y1 = jnp.concatenate([y1_sc.reshape(SC_COLS // ROW, ROW), y1_tc], axis=0)
    out_t = _make_lookup(batch, n_prows)(user_idx.astype(jnp.int32), y0, y1)
    return out_t.T


# speedup vs baseline: 1.6908x; 1.6908x over previous
import functools

import jax
import jax.numpy as jnp
from jax import lax
from jax.experimental import pallas as pl
from jax.experimental.pallas import tpu as pltpu
from jax.experimental.pallas import tpu_sc as plsc

EMBED_DIM = 64
N_ACTION = 2
N_CORES = 2
N_SUBCORES = 16
NW = N_CORES * N_SUBCORES
LANES = 16
ROW = 128
RBLK = 64
CBLK = RBLK * ROW
COLS_PER_TEC = 4096
SC_COLS = COLS_PER_TEC * NW
CCH = 512
N_CH = COLS_PER_TEC // CCH
CHUNK = 256


def _proj_body(w_ref, b_ref, x_ref, o0_ref, o1_ref):
    y = lax.dot_general(
        w_ref[...], x_ref[...],
        (((1,), (0,)), ((), ())),
        preferred_element_type=jnp.float32,
        precision=lax.Precision.HIGHEST,
    ) + b_ref[...]
    o0_ref[...] = y[0:1, :].reshape(RBLK, ROW)
    o1_ref[...] = y[1:2, :].reshape(RBLK, ROW)


@functools.lru_cache(maxsize=None)
def _make_tc_proj(n_rows: int):
    n_prows = (n_rows + ROW - 1) // ROW - SC_COLS // ROW
    blk0 = SC_COLS // CBLK
    grid = (n_prows * ROW + CBLK - 1) // CBLK
    return pl.pallas_call(
        _proj_body,
        grid=(grid,),
        in_specs=[
            pl.BlockSpec((N_ACTION, EMBED_DIM), lambda i: (0, 0)),
            pl.BlockSpec((N_ACTION, 1), lambda i: (0, 0)),
            pl.BlockSpec((EMBED_DIM, CBLK), lambda i: (0, blk0 + i)),
        ],
        out_specs=[
            pl.BlockSpec((RBLK, ROW), lambda i: (i, 0)),
            pl.BlockSpec((RBLK, ROW), lambda i: (i, 0)),
        ],
        out_shape=[
            jax.ShapeDtypeStruct((n_prows, ROW), jnp.float32),
            jax.ShapeDtypeStruct((n_prows, ROW), jnp.float32),
        ],
    )


@functools.lru_cache(maxsize=None)
def _make_sc_proj():
    mesh = plsc.VectorSubcoreMesh(core_axis_name="c", subcore_axis_name="s")

    @functools.partial(
        pl.kernel,
        mesh=mesh,
        out_type=[
            jax.ShapeDtypeStruct((SC_COLS,), jnp.float32),
            jax.ShapeDtypeStruct((SC_COLS,), jnp.float32),
        ],
        scratch_types=[
            pltpu.VMEM((EMBED_DIM // 8, 8, CCH), jnp.float32),
            pltpu.VMEM((EMBED_DIM // 8, 8, CCH), jnp.float32),
            pltpu.VMEM((256,), jnp.float32),
            pltpu.VMEM((COLS_PER_TEC,), jnp.float32),
            pltpu.VMEM((COLS_PER_TEC,), jnp.float32),
            pltpu.SemaphoreType.DMA,
            pltpu.SemaphoreType.DMA,
        ],
        compiler_params=pltpu.CompilerParams(
            use_tc_tiling_on_sc=True, needs_layout_passes=False
        ),
    )
    def sc_proj(table_hbm, wb_hbm, y0_hbm, y1_hbm,
                buf_a, buf_b, wb_v, out0_v, out1_v, sem_a, sem_b):
        wid = lax.axis_index("s") * N_CORES + lax.axis_index("c")
        cbase = wid * COLS_PER_TEC
        pltpu.sync_copy(wb_hbm, wb_v)
        w_vecs = [
            [wb_v[pl.ds(a * EMBED_DIM + k * LANES, LANES)] for k in range(4)]
            for a in range(N_ACTION)
        ]
        bias_vec = wb_v[pl.ds(2 * EMBED_DIM, LANES)]
        bufs = (buf_a, buf_b)
        sems = (sem_a, sem_b)

        for b in range(2):
            pltpu.async_copy(
                table_hbm.at[:, :, pl.ds(cbase + b * CCH, CCH)],
                bufs[b], sems[b],
            )

        def outer(c2, _):
            for b in range(2):
                c = c2 * 2 + b
                buf, sem = bufs[b], sems[b]
                pltpu.make_async_copy(
                    table_hbm.at[:, :, pl.ds(0, CCH)], buf, sem
                ).wait()

                def blk_body(jb, _):
                    off = jb * LANES
                    acc = [jnp.full((LANES,), bias_vec[a], jnp.float32)
                           for a in range(N_ACTION)]
                    for d in range(EMBED_DIM):
                        val = buf[d // 8, d % 8, pl.ds(off, LANES)]
                        for a in range(N_ACTION):
                            acc[a] = acc[a] + val * w_vecs[a][d // 16][d % 16]
                    out0_v[pl.ds(c * CCH + off, LANES)] = acc[0]
                    out1_v[pl.ds(c * CCH + off, LANES)] = acc[1]
                    return 0

                lax.fori_loop(0, CCH // LANES, blk_body, 0)

                @pl.when(c + 2 < N_CH)
                def _():
                    pltpu.async_copy(
                        table_hbm.at[:, :, pl.ds(cbase + (c + 2) * CCH, CCH)],
                        buf, sem,
                    )
            return 0

        lax.fori_loop(0, N_CH // 2, outer, 0)
        pltpu.sync_copy(out0_v, y0_hbm.at[pl.ds(cbase, COLS_PER_TEC)])
        pltpu.sync_copy(out1_v, y1_hbm.at[pl.ds(cbase, COLS_PER_TEC)])

    return sc_proj


@functools.lru_cache(maxsize=None)
def _make_lookup(batch: int, n_prows: int):
    assert batch % (CHUNK * NW) == 0
    b_per_w = batch // NW
    n_chunks = b_per_w // CHUNK
    mesh = plsc.VectorSubcoreMesh(core_axis_name="c", subcore_axis_name="s")

    @functools.partial(
        pl.kernel,
        mesh=mesh,
        out_type=jax.ShapeDtypeStruct((N_ACTION, batch), jnp.float32),
        scratch_types=[
            pltpu.VMEM((b_per_w,), jnp.int32),
            pltpu.VMEM((b_per_w,), jnp.int32),
            pltpu.VMEM((CHUNK, ROW), jnp.float32),
            pltpu.VMEM((CHUNK, ROW), jnp.float32),
            pltpu.VMEM((b_per_w,), jnp.float32),
            pltpu.VMEM((b_per_w,), jnp.float32),
            pltpu.SemaphoreType.DMA,
        ],
        compiler_params=pltpu.CompilerParams(
            use_tc_tiling_on_sc=True, needs_layout_passes=False
        ),
    )
    def lookup(idx_hbm, y0_hbm, y1_hbm, out_hbm,
               idx_v, row_v, g0_v, g1_v, out0_v, out1_v, sem):
        wid = lax.axis_index("s") * N_CORES + lax.axis_index("c")
        base = wid * b_per_w
        pltpu.sync_copy(idx_hbm.at[pl.ds(base, b_per_w)], idx_v)

        def split_body(k, _):
            v = idx_v[pl.ds(k * LANES, LANES)]
            row_v[pl.ds(k * LANES, LANES)] = v >> 7
            return 0

        lax.fori_loop(0, b_per_w // LANES, split_body, 0)
        lane_iota = lax.iota(jnp.int32, LANES)

        def chunk_body(c, _):
            rows = row_v.at[pl.ds(c * CHUNK, CHUNK)]
            cp0 = pltpu.async_copy(y0_hbm.at[rows], g0_v, sem)
            cp1 = pltpu.async_copy(y1_hbm.at[rows], g1_v, sem)
            cp0.wait()
            cp1.wait()

            def blk_body(jb, _):
                iv = idx_v[pl.ds(c * CHUNK + jb * LANES, LANES)]
                slot = jb * LANES + lane_iota
                col = iv & (ROW - 1)
                v0 = plsc.load_gather(g0_v, [slot, col])
                v1 = plsc.load_gather(g1_v, [slot, col])
                out0_v[pl.ds(c * CHUNK + jb * LANES, LANES)] = v0
                out1_v[pl.ds(c * CHUNK + jb * LANES, LANES)] = v1
                return 0

            lax.fori_loop(0, CHUNK // LANES, blk_body, 0)
            return 0

        lax.fori_loop(0, n_chunks, chunk_body, 0)
        pltpu.sync_copy(out0_v, out_hbm.at[0, pl.ds(base, b_per_w)])
        pltpu.sync_copy(out1_v, out_hbm.at[1, pl.ds(base, b_per_w)])

    return lookup


def kernel(user_idx, table, W, b):
    batch = user_idx.shape[0]
    n_rows = table.shape[0]
    n_prows = (n_rows + ROW - 1) // ROW
    table_t = table.T
    table_t3 = table_t.reshape(EMBED_DIM // 8, 8, n_rows)
    wb = jnp.zeros((256,), jnp.float32)
    wb = wb.at[: N_ACTION * EMBED_DIM].set(W.reshape(-1))
    wb = wb.at[2 * EMBED_DIM : 2 * EMBED_DIM + N_ACTION].set(b)

    y0_sc, y1_sc = _make_sc_proj()(table_t3, wb)
    y0_tc, y1_tc = _make_tc_proj(n_rows)(W, b.reshape(N_ACTION, 1), table_t)
    y0 = jnp.concatenate([y0_sc.reshape(SC_COLS // ROW, ROW), y0_tc], axis=0)
    y1 = jnp.concatenate([y1_sc.reshape(SC_COLS // ROW, ROW), y1_tc], axis=0)
    out_t = _make_lookup(batch, n_prows)(user_idx.astype(jnp.int32), y0, y1)
    return out_t.T


# speedup vs baseline: 1.7507x; 1.0355x over previous
import functools

import jax
import jax.numpy as jnp
from jax import lax
from jax.experimental import pallas as pl
from jax.experimental.pallas import tpu as pltpu
from jax.experimental.pallas import tpu_sc as plsc

EMBED_DIM = 64
N_ACTION = 2
N_CORES = 2
N_SUBCORES = 16
NW = N_CORES * N_SUBCORES
LANES = 16
ROW = 128
RBLK = 80
CHUNK = 256


def _proj_body(w_ref, b_ref, x_ref, o0_ref, o1_ref):
    y = lax.dot_general(
        w_ref[...], x_ref[...],
        (((1,), (0,)), ((), ())),
        preferred_element_type=jnp.float32,
        precision=lax.Precision.HIGHEST,
    ) + b_ref[...]
    o0_ref[...] = y[0:1, :].reshape(RBLK, ROW)
    o1_ref[...] = y[1:2, :].reshape(RBLK, ROW)


@functools.lru_cache(maxsize=None)
def _make_proj(n_rows: int):
    n_prows = (n_rows + ROW - 1) // ROW
    grid = (n_prows + RBLK - 1) // RBLK
    cblk = RBLK * ROW
    return pl.pallas_call(
        _proj_body,
        grid=(grid,),
        in_specs=[
            pl.BlockSpec((N_ACTION, EMBED_DIM), lambda i: (0, 0)),
            pl.BlockSpec((N_ACTION, 1), lambda i: (0, 0)),
            pl.BlockSpec((EMBED_DIM, cblk), lambda i: (0, i)),
        ],
        out_specs=[
            pl.BlockSpec((RBLK, ROW), lambda i: (i, 0)),
            pl.BlockSpec((RBLK, ROW), lambda i: (i, 0)),
        ],
        out_shape=[
            jax.ShapeDtypeStruct((n_prows, ROW), jnp.float32),
            jax.ShapeDtypeStruct((n_prows, ROW), jnp.float32),
        ],
    )


@functools.lru_cache(maxsize=None)
def _make_lookup(batch: int, n_prows: int):
    assert batch % (CHUNK * NW) == 0
    b_per_w = batch // NW
    n_chunks = b_per_w // CHUNK
    mesh = plsc.VectorSubcoreMesh(core_axis_name="c", subcore_axis_name="s")

    @functools.partial(
        pl.kernel,
        mesh=mesh,
        out_type=jax.ShapeDtypeStruct((N_ACTION, batch), jnp.float32),
        scratch_types=[
            pltpu.VMEM((b_per_w,), jnp.int32),
            pltpu.VMEM((b_per_w,), jnp.int32),
            pltpu.VMEM((CHUNK, ROW), jnp.float32),
            pltpu.VMEM((CHUNK, ROW), jnp.float32),
            pltpu.VMEM((b_per_w,), jnp.float32),
            pltpu.VMEM((b_per_w,), jnp.float32),
            pltpu.SemaphoreType.DMA,
        ],
        compiler_params=pltpu.CompilerParams(
            use_tc_tiling_on_sc=True, needs_layout_passes=False
        ),
    )
    def lookup(idx_hbm, y0_hbm, y1_hbm, out_hbm,
               idx_v, row_v, g0_v, g1_v, out0_v, out1_v, sem):
        wid = lax.axis_index("s") * N_CORES + lax.axis_index("c")
        base = wid * b_per_w
        pltpu.sync_copy(idx_hbm.at[pl.ds(base, b_per_w)], idx_v)

        def split_body(k, _):
            v = idx_v[pl.ds(k * LANES, LANES)]
            row_v[pl.ds(k * LANES, LANES)] = v >> 7
            return 0

        lax.fori_loop(0, b_per_w // LANES, split_body, 0)
        lane_iota = lax.iota(jnp.int32, LANES)

        def chunk_body(c, _):
            rows = row_v.at[pl.ds(c * CHUNK, CHUNK)]
            cp0 = pltpu.async_copy(y0_hbm.at[rows], g0_v, sem)
            cp1 = pltpu.async_copy(y1_hbm.at[rows], g1_v, sem)
            cp0.wait()
            cp1.wait()

            def blk_body(jb, _):
                iv = idx_v[pl.ds(c * CHUNK + jb * LANES, LANES)]
                slot = jb * LANES + lane_iota
                col = iv & (ROW - 1)
                v0 = plsc.load_gather(g0_v, [slot, col])
                v1 = plsc.load_gather(g1_v, [slot, col])
                out0_v[pl.ds(c * CHUNK + jb * LANES, LANES)] = v0
                out1_v[pl.ds(c * CHUNK + jb * LANES, LANES)] = v1
                return 0

            lax.fori_loop(0, CHUNK // LANES, blk_body, 0)
            return 0

        lax.fori_loop(0, n_chunks, chunk_body, 0)
        pltpu.sync_copy(out0_v, out_hbm.at[0, pl.ds(base, b_per_w)])
        pltpu.sync_copy(out1_v, out_hbm.at[1, pl.ds(base, b_per_w)])

    return lookup


def kernel(user_idx, table, W, b):
    batch = user_idx.shape[0]
    n_rows = table.shape[0]
    n_prows = (n_rows + ROW - 1) // ROW
    table_t = table.T
    y0, y1 = _make_proj(n_rows)(W, b.reshape(N_ACTION, 1), table_t)
    out_t = _make_lookup(batch, n_prows)(user_idx.astype(jnp.int32), y0, y1)
    return out_t.T


# speedup vs baseline: 1.7535x; 1.0016x over previous
import functools

import jax
import jax.numpy as jnp
from jax import lax
from jax.experimental import pallas as pl
from jax.experimental.pallas import tpu as pltpu
from jax.experimental.pallas import tpu_sc as plsc

EMBED_DIM = 64
N_ACTION = 2
N_CORES = 2
N_SUBCORES = 16
NW = N_CORES * N_SUBCORES
LANES = 16
ROW = 128
RBLK = 80
CHUNK = 128


def _proj_body(w_ref, b_ref, x_ref, o0_ref, o1_ref):
    y = lax.dot_general(
        w_ref[...], x_ref[...],
        (((1,), (0,)), ((), ())),
        preferred_element_type=jnp.float32,
        precision=lax.Precision.HIGHEST,
    ) + b_ref[...]
    o0_ref[...] = y[0:1, :].reshape(RBLK, ROW)
    o1_ref[...] = y[1:2, :].reshape(RBLK, ROW)


@functools.lru_cache(maxsize=None)
def _make_proj(n_rows: int):
    n_prows = (n_rows + ROW - 1) // ROW
    grid = (n_prows + RBLK - 1) // RBLK
    cblk = RBLK * ROW
    return pl.pallas_call(
        _proj_body,
        grid=(grid,),
        in_specs=[
            pl.BlockSpec((N_ACTION, EMBED_DIM), lambda i: (0, 0)),
            pl.BlockSpec((N_ACTION, 1), lambda i: (0, 0)),
            pl.BlockSpec((EMBED_DIM, cblk), lambda i: (0, i)),
        ],
        out_specs=[
            pl.BlockSpec((RBLK, ROW), lambda i: (i, 0)),
            pl.BlockSpec((RBLK, ROW), lambda i: (i, 0)),
        ],
        out_shape=[
            jax.ShapeDtypeStruct((n_prows, ROW), jnp.float32),
            jax.ShapeDtypeStruct((n_prows, ROW), jnp.float32),
        ],
    )


@functools.lru_cache(maxsize=None)
def _make_lookup(batch: int, n_prows: int):
    assert batch % (CHUNK * NW) == 0
    b_per_w = batch // NW
    n_chunks = b_per_w // CHUNK
    mesh = plsc.VectorSubcoreMesh(core_axis_name="c", subcore_axis_name="s")

    @functools.partial(
        pl.kernel,
        mesh=mesh,
        out_type=jax.ShapeDtypeStruct((N_ACTION, batch), jnp.float32),
        scratch_types=[
            pltpu.VMEM((b_per_w,), jnp.int32),
            pltpu.VMEM((b_per_w,), jnp.int32),
            pltpu.VMEM((CHUNK, ROW), jnp.float32),
            pltpu.VMEM((CHUNK, ROW), jnp.float32),
            pltpu.VMEM((CHUNK, ROW), jnp.float32),
            pltpu.VMEM((CHUNK, ROW), jnp.float32),
            pltpu.VMEM((b_per_w,), jnp.float32),
            pltpu.VMEM((b_per_w,), jnp.float32),
            pltpu.SemaphoreType.DMA,
            pltpu.SemaphoreType.DMA,
        ],
        compiler_params=pltpu.CompilerParams(
            use_tc_tiling_on_sc=True, needs_layout_passes=False
        ),
    )
    def lookup(idx_hbm, y0_hbm, y1_hbm, out_hbm,
               idx_v, row_v, ga0_v, ga1_v, gb0_v, gb1_v,
               out0_v, out1_v, sem_a, sem_b):
        wid = lax.axis_index("s") * N_CORES + lax.axis_index("c")
        base = wid * b_per_w
        pltpu.sync_copy(idx_hbm.at[pl.ds(base, b_per_w)], idx_v)

        def split_body(k, _):
            v = idx_v[pl.ds(k * LANES, LANES)]
            row_v[pl.ds(k * LANES, LANES)] = v >> 7
            return 0

        lax.fori_loop(0, b_per_w // LANES, split_body, 0)
        lane_iota = lax.iota(jnp.int32, LANES)
        bufs = ((ga0_v, ga1_v, sem_a), (gb0_v, gb1_v, sem_b))

        def fire(c, g0, g1, sem):
            rows = row_v.at[pl.ds(c * CHUNK, CHUNK)]
            pltpu.async_copy(y0_hbm.at[rows], g0, sem)
            pltpu.async_copy(y1_hbm.at[rows], g1, sem)

        for b in range(2):
            fire(jnp.int32(b), *bufs[b])

        def outer(c2, _):
            for b in range(2):
                c = c2 * 2 + b
                g0, g1, sem = bufs[b]
                for g in (g0, g1):
                    pltpu.make_async_copy(
                        y0_hbm.at[pl.ds(0, CHUNK)], g, sem
                    ).wait()

                def blk_body(jb, _):
                    iv = idx_v[pl.ds(c * CHUNK + jb * LANES, LANES)]
                    slot = jb * LANES + lane_iota
                    col = iv & (ROW - 1)
                    v0 = plsc.load_gather(g0, [slot, col])
                    v1 = plsc.load_gather(g1, [slot, col])
                    out0_v[pl.ds(c * CHUNK + jb * LANES, LANES)] = v0
                    out1_v[pl.ds(c * CHUNK + jb * LANES, LANES)] = v1
                    return 0

                lax.fori_loop(0, CHUNK // LANES, blk_body, 0)

                @pl.when(c + 2 < n_chunks)
                def _():
                    fire(c + 2, g0, g1, sem)
            return 0

        lax.fori_loop(0, n_chunks // 2, outer, 0)
        pltpu.sync_copy(out0_v, out_hbm.at[0, pl.ds(base, b_per_w)])
        pltpu.sync_copy(out1_v, out_hbm.at[1, pl.ds(base, b_per_w)])

    return lookup


def kernel(user_idx, table, W, b):
    batch = user_idx.shape[0]
    n_rows = table.shape[0]
    n_prows = (n_rows + ROW - 1) // ROW
    table_t = table.T
    y0, y1 = _make_proj(n_rows)(W, b.reshape(N_ACTION, 1), table_t)
    out_t = _make_lookup(batch, n_prows)(user_idx.astype(jnp.int32), y0, y1)
    return out_t.T
